# baseline (device time: 1366547 ns/iter reference)
import jax
import jax.numpy as jnp
from jax import lax
from jax.experimental import pallas as pl
from jax.experimental.pallas import tpu as pltpu

S = 2048
N = 8192
K_LOC = 4096
S_OUT = 1024
CB = 1024
NB = N // CB
BK = 512
KB = K_LOC // BK
NSEND = NB // 2

MESH = pl.DeviceIdType.MESH


def _fused_body(a_ref, b_ref, p_ref, po_ref,
                stage, x_send, x_recv, y_send, y_recv):
    n = pl.program_id(0)
    k = pl.program_id(1)
    my_x = lax.axis_index("x")
    my_y = lax.axis_index("y")
    x_partner = (1 - my_x, my_y)
    y_partner = (my_x, 1 - my_y)

    @pl.when((n == 0) & (k == 0))
    def _():
        bar = pltpu.get_barrier_semaphore()
        pl.semaphore_signal(bar, inc=1, device_id=x_partner,
                            device_id_type=MESH)
        pl.semaphore_signal(bar, inc=1, device_id=y_partner,
                            device_id_type=MESH)
        pl.semaphore_wait(bar, 2)

    @pl.when(k == 0)
    def _():
        p_ref[...] = jnp.zeros_like(p_ref)

    p_ref[...] += lax.dot_general(
        a_ref[...], b_ref[...], (((1,), (0,)), ((), ())),
        preferred_element_type=jnp.float32)

    def x_desc(jj, blk):
        return pltpu.make_async_remote_copy(
            src_ref=stage.at[jj],
            dst_ref=po_ref.at[:, pl.ds(blk * CB, CB)],
            send_sem=x_send.at[jj],
            recv_sem=x_recv.at[jj],
            device_id=x_partner,
            device_id_type=MESH)

    def y_out_desc(jj):
        blk = 2 * jj + my_y
        return pltpu.make_async_remote_copy(
            src_ref=po_ref.at[:, pl.ds(blk * CB, CB)],
            dst_ref=po_ref.at[:, pl.ds(blk * CB, CB)],
            send_sem=y_send.at[jj],
            recv_sem=y_recv.at[jj],
            device_id=y_partner,
            device_id_type=MESH)

    def y_in_desc(jj):
        blk = 2 * jj + (1 - my_y)
        return pltpu.make_async_remote_copy(
            src_ref=po_ref.at[:, pl.ds(blk * CB, CB)],
            dst_ref=po_ref.at[:, pl.ds(blk * CB, CB)],
            send_sem=y_send.at[jj],
            recv_sem=y_recv.at[jj],
            device_id=y_partner,
            device_id_type=MESH)

    k_last = KB - 1

    @pl.when((k == k_last) & (n % 2 == my_y))
    def _():
        jj = n // 2
        stage[jj] = p_ref[pl.ds((1 - my_x) * S_OUT, S_OUT), :]
        x_desc(jj, n).start()

    @pl.when((k == k_last) & (n >= 3) & (n % 2 == 1))
    def _():
        jj = (n - 3) // 2
        x_desc(jj, 2 * jj + my_y).wait_recv()
        y_out_desc(jj).start()

    @pl.when((k == k_last) & (n == NB - 1))
    def _():
        jj = NSEND - 1
        x_desc(jj, 2 * jj + my_y).wait_recv()
        y_out_desc(jj).start()
        for t in range(NSEND):
            x_desc(t, 2 * t + my_y).wait_send()
        for t in range(NSEND):
            y_out_desc(t).wait_send()
        for t in range(NSEND):
            y_in_desc(t).wait_recv()


def _relayout_body(o_ref, a2_ref, sems):
    for h in range(32):
        pltpu.make_async_copy(
            o_ref.at[:, h, :], a2_ref.at[:, pl.ds(h * 128, 128)], sems.at[h]
        ).start()
    for h in range(32):
        pltpu.make_async_copy(
            o_ref.at[:, h, :], a2_ref.at[:, pl.ds(h * 128, 128)], sems.at[h]
        ).wait()


def _add_body(p_hbm, po_ref, out_ref, keep, sem):
    j = pl.program_id(0)
    my_x = lax.axis_index("x")
    cp = pltpu.make_async_copy(
        p_hbm.at[pl.ds(my_x * S_OUT, S_OUT), pl.ds(j * CB, CB)], keep, sem)
    cp.start()
    cp.wait()
    out_ref[...] = keep[...] + po_ref[...]


def kernel(O, Wo):
    b, s, h, d = O.shape
    o3 = O.reshape(s, h, d)
    a2 = pl.pallas_call(
        _relayout_body,
        in_specs=[pl.BlockSpec(memory_space=pl.ANY)],
        out_specs=pl.BlockSpec(memory_space=pl.ANY),
        out_shape=jax.ShapeDtypeStruct((S, K_LOC), jnp.float32),
        scratch_shapes=[pltpu.SemaphoreType.DMA((32,))],
    )(o3)

    p, po = pl.pallas_call(
        _fused_body,
        grid=(NB, KB),
        in_specs=[
            pl.BlockSpec((S, BK), lambda n, k: (0, k)),
            pl.BlockSpec((BK, CB), lambda n, k: (k, n)),
        ],
        out_specs=[
            pl.BlockSpec((S, CB), lambda n, k: (0, n)),
            pl.BlockSpec(memory_space=pl.ANY),
        ],
        out_shape=[
            jax.ShapeDtypeStruct((S, N), jnp.float32),
            jax.ShapeDtypeStruct((S_OUT, N), jnp.float32),
        ],
        scratch_shapes=[
            pltpu.VMEM((NSEND, S_OUT, CB), jnp.float32),
            pltpu.SemaphoreType.DMA((NSEND,)),
            pltpu.SemaphoreType.DMA((NSEND,)),
            pltpu.SemaphoreType.DMA((NSEND,)),
            pltpu.SemaphoreType.DMA((NSEND,)),
        ],
        compiler_params=pltpu.CompilerParams(
            dimension_semantics=("arbitrary", "arbitrary"),
            vmem_limit_bytes=56 * 1024 * 1024,
            collective_id=0),
    )(a2, Wo)

    out = pl.pallas_call(
        _add_body,
        grid=(NB,),
        in_specs=[
            pl.BlockSpec(memory_space=pl.ANY),
            pl.BlockSpec((S_OUT, CB), lambda j: (0, j)),
        ],
        out_specs=pl.BlockSpec((S_OUT, CB), lambda j: (0, j)),
        out_shape=jax.ShapeDtypeStruct((S_OUT, N), jnp.float32),
        scratch_shapes=[
            pltpu.VMEM((S_OUT, CB), jnp.float32),
            pltpu.SemaphoreType.DMA,
        ],
    )(p, po)
    return out.reshape(1, S_OUT, N)


# device time: 370169 ns/iter; 3.6917x vs baseline; 3.6917x over previous
import jax
import jax.numpy as jnp
from jax import lax
from jax.experimental import pallas as pl
from jax.experimental.pallas import tpu as pltpu

S = 2048
N = 8192
K_LOC = 4096
S_OUT = 1024
CB = 1024
NB = N // CB
BK = 512
KB = K_LOC // BK
NSEND = NB // 2

MESH = pl.DeviceIdType.MESH


def _fused_body(a_ref, b_ref, p_ref, po_ref,
                stage, x_send, x_recv, y_send, y_recv):
    n = pl.program_id(0)
    k = pl.program_id(1)
    my_x = lax.axis_index("x")
    my_y = lax.axis_index("y")
    x_partner = (1 - my_x, my_y)
    y_partner = (my_x, 1 - my_y)

    @pl.when((n == 0) & (k == 0))
    def _():
        bar = pltpu.get_barrier_semaphore()
        pl.semaphore_signal(bar, inc=1, device_id=x_partner,
                            device_id_type=MESH)
        pl.semaphore_signal(bar, inc=1, device_id=y_partner,
                            device_id_type=MESH)
        pl.semaphore_wait(bar, 2)

    @pl.when(k == 0)
    def _():
        p_ref[...] = jnp.zeros_like(p_ref)

    p_ref[...] += lax.dot_general(
        a_ref[...], b_ref[...], (((1,), (0,)), ((), ())),
        preferred_element_type=jnp.float32)

    def x_desc(jj, blk):
        return pltpu.make_async_remote_copy(
            src_ref=stage.at[jj],
            dst_ref=po_ref.at[:, pl.ds(blk * CB, CB)],
            send_sem=x_send.at[jj],
            recv_sem=x_recv.at[jj],
            device_id=x_partner,
            device_id_type=MESH)

    def y_out_desc(jj):
        blk = 2 * jj + my_y
        return pltpu.make_async_remote_copy(
            src_ref=po_ref.at[:, pl.ds(blk * CB, CB)],
            dst_ref=po_ref.at[:, pl.ds(blk * CB, CB)],
            send_sem=y_send.at[jj],
            recv_sem=y_recv.at[jj],
            device_id=y_partner,
            device_id_type=MESH)

    def y_in_desc(jj):
        blk = 2 * jj + (1 - my_y)
        return pltpu.make_async_remote_copy(
            src_ref=po_ref.at[:, pl.ds(blk * CB, CB)],
            dst_ref=po_ref.at[:, pl.ds(blk * CB, CB)],
            send_sem=y_send.at[jj],
            recv_sem=y_recv.at[jj],
            device_id=y_partner,
            device_id_type=MESH)

    k_last = KB - 1

    @pl.when((k == k_last) & (n % 2 == my_y))
    def _():
        jj = n // 2
        stage[jj] = p_ref[pl.ds((1 - my_x) * S_OUT, S_OUT), :]
        x_desc(jj, n).start()

    @pl.when((k == k_last) & (n >= 3) & (n % 2 == 1))
    def _():
        jj = (n - 3) // 2
        x_desc(jj, 2 * jj + my_y).wait_recv()
        y_out_desc(jj).start()

    @pl.when((k == k_last) & (n == NB - 1))
    def _():
        jj = NSEND - 1
        x_desc(jj, 2 * jj + my_y).wait_recv()
        y_out_desc(jj).start()
        for t in range(NSEND):
            x_desc(t, 2 * t + my_y).wait_send()
        for t in range(NSEND):
            y_out_desc(t).wait_send()
        for t in range(NSEND):
            y_in_desc(t).wait_recv()


def _relayout_body(o_ref, a2_ref):
    sb = o_ref.shape[0]
    a2_ref[...] = o_ref[...].reshape(sb, K_LOC)


def _add_body(p_hbm, po_ref, out_ref, keep, sem):
    j = pl.program_id(0)
    my_x = lax.axis_index("x")
    cp = pltpu.make_async_copy(
        p_hbm.at[pl.ds(my_x * S_OUT, S_OUT), pl.ds(j * CB, CB)], keep, sem)
    cp.start()
    cp.wait()
    out_ref[...] = keep[...] + po_ref[...]


def kernel(O, Wo):
    b, s, h, d = O.shape
    o3 = O.reshape(s, h, d)
    a2 = pl.pallas_call(
        _relayout_body,
        grid=(8,),
        in_specs=[pl.BlockSpec((S // 8, h, d), lambda i: (i, 0, 0))],
        out_specs=pl.BlockSpec((S // 8, K_LOC), lambda i: (i, 0)),
        out_shape=jax.ShapeDtypeStruct((S, K_LOC), jnp.float32),
    )(o3)

    p, po = pl.pallas_call(
        _fused_body,
        grid=(NB, KB),
        in_specs=[
            pl.BlockSpec((S, BK), lambda n, k: (0, k)),
            pl.BlockSpec((BK, CB), lambda n, k: (k, n)),
        ],
        out_specs=[
            pl.BlockSpec((S, CB), lambda n, k: (0, n)),
            pl.BlockSpec(memory_space=pl.ANY),
        ],
        out_shape=[
            jax.ShapeDtypeStruct((S, N), jnp.float32),
            jax.ShapeDtypeStruct((S_OUT, N), jnp.float32),
        ],
        scratch_shapes=[
            pltpu.VMEM((NSEND, S_OUT, CB), jnp.float32),
            pltpu.SemaphoreType.DMA((NSEND,)),
            pltpu.SemaphoreType.DMA((NSEND,)),
            pltpu.SemaphoreType.DMA((NSEND,)),
            pltpu.SemaphoreType.DMA((NSEND,)),
        ],
        compiler_params=pltpu.CompilerParams(
            dimension_semantics=("arbitrary", "arbitrary"),
            vmem_limit_bytes=56 * 1024 * 1024,
            collective_id=0),
    )(a2, Wo)

    out = pl.pallas_call(
        _add_body,
        grid=(NB,),
        in_specs=[
            pl.BlockSpec(memory_space=pl.ANY),
            pl.BlockSpec((S_OUT, CB), lambda j: (0, j)),
        ],
        out_specs=pl.BlockSpec((S_OUT, CB), lambda j: (0, j)),
        out_shape=jax.ShapeDtypeStruct((S_OUT, N), jnp.float32),
        scratch_shapes=[
            pltpu.VMEM((S_OUT, CB), jnp.float32),
            pltpu.SemaphoreType.DMA,
        ],
    )(p, po)
    return out.reshape(1, S_OUT, N)


# device time: 337470 ns/iter; 4.0494x vs baseline; 1.0969x over previous
import jax
import jax.numpy as jnp
from jax import lax
from jax.experimental import pallas as pl
from jax.experimental.pallas import tpu as pltpu

S = 2048
N = 8192
K_LOC = 4096
S_OUT = 1024
CB = 1024
NB = N // CB
BK = 512
KB = K_LOC // BK
NSEND = NB // 2

MESH = pl.DeviceIdType.MESH


def _fused_body(a_ref, b_ref, p_ref, po_ref,
                stage, x_send, x_recv, y_send, y_recv):
    n = pl.program_id(0)
    k = pl.program_id(1)
    my_x = lax.axis_index("x")
    my_y = lax.axis_index("y")
    x_partner = (1 - my_x, my_y)
    y_partner = (my_x, 1 - my_y)

    @pl.when((n == 0) & (k == 0))
    def _():
        bar = pltpu.get_barrier_semaphore()
        pl.semaphore_signal(bar, inc=1, device_id=x_partner,
                            device_id_type=MESH)
        pl.semaphore_signal(bar, inc=1, device_id=y_partner,
                            device_id_type=MESH)
        pl.semaphore_wait(bar, 2)

    is_sender = n % 2 == my_y
    jj = n // 2

    @pl.when(k == 0)
    def _():
        p_ref[...] = jnp.zeros_like(p_ref)

    a_keep = a_ref[pl.ds(my_x * S_OUT, S_OUT), :]
    p_ref[...] += lax.dot_general(
        a_keep, b_ref[...], (((1,), (0,)), ((), ())),
        preferred_element_type=jnp.float32)

    @pl.when(is_sender & (k == 0))
    def _():
        stage[jj] = jnp.zeros_like(stage[jj])

    @pl.when(is_sender)
    def _():
        a_send = a_ref[pl.ds((1 - my_x) * S_OUT, S_OUT), :]
        stage[jj] += lax.dot_general(
            a_send, b_ref[...], (((1,), (0,)), ((), ())),
            preferred_element_type=jnp.float32)

    def x_desc(jj, blk):
        return pltpu.make_async_remote_copy(
            src_ref=stage.at[jj],
            dst_ref=po_ref.at[:, pl.ds(blk * CB, CB)],
            send_sem=x_send.at[jj],
            recv_sem=x_recv.at[jj],
            device_id=x_partner,
            device_id_type=MESH)

    def y_out_desc(jj):
        blk = 2 * jj + my_y
        return pltpu.make_async_remote_copy(
            src_ref=po_ref.at[:, pl.ds(blk * CB, CB)],
            dst_ref=po_ref.at[:, pl.ds(blk * CB, CB)],
            send_sem=y_send.at[jj],
            recv_sem=y_recv.at[jj],
            device_id=y_partner,
            device_id_type=MESH)

    def y_in_desc(jj):
        blk = 2 * jj + (1 - my_y)
        return pltpu.make_async_remote_copy(
            src_ref=po_ref.at[:, pl.ds(blk * CB, CB)],
            dst_ref=po_ref.at[:, pl.ds(blk * CB, CB)],
            send_sem=y_send.at[jj],
            recv_sem=y_recv.at[jj],
            device_id=y_partner,
            device_id_type=MESH)

    k_last = KB - 1

    @pl.when((k == k_last) & is_sender)
    def _():
        x_desc(jj, n).start()

    @pl.when((k == k_last) & (n >= 3) & (n % 2 == 1))
    def _():
        jj = (n - 3) // 2
        x_desc(jj, 2 * jj + my_y).wait_recv()
        y_out_desc(jj).start()

    @pl.when((k == k_last) & (n == NB - 1))
    def _():
        jj = NSEND - 1
        x_desc(jj, 2 * jj + my_y).wait_recv()
        y_out_desc(jj).start()
        for t in range(NSEND):
            x_desc(t, 2 * t + my_y).wait_send()
        for t in range(NSEND):
            y_out_desc(t).wait_send()
        for t in range(NSEND):
            y_in_desc(t).wait_recv()


def _relayout_body(o_ref, a2_ref):
    sb = o_ref.shape[0]
    a2_ref[...] = o_ref[...].reshape(sb, K_LOC)


def _add_body(p_ref, po_ref, out_ref):
    out_ref[...] = p_ref[...] + po_ref[...]


def kernel(O, Wo):
    b, s, h, d = O.shape
    o3 = O.reshape(s, h, d)
    a2 = pl.pallas_call(
        _relayout_body,
        grid=(8,),
        in_specs=[pl.BlockSpec((S // 8, h, d), lambda i: (i, 0, 0))],
        out_specs=pl.BlockSpec((S // 8, K_LOC), lambda i: (i, 0)),
        out_shape=jax.ShapeDtypeStruct((S, K_LOC), jnp.float32),
    )(o3)

    p, po = pl.pallas_call(
        _fused_body,
        grid=(NB, KB),
        in_specs=[
            pl.BlockSpec((S, BK), lambda n, k: (0, k)),
            pl.BlockSpec((BK, CB), lambda n, k: (k, n)),
        ],
        out_specs=[
            pl.BlockSpec((S_OUT, CB), lambda n, k: (0, n)),
            pl.BlockSpec(memory_space=pl.ANY),
        ],
        out_shape=[
            jax.ShapeDtypeStruct((S_OUT, N), jnp.float32),
            jax.ShapeDtypeStruct((S_OUT, N), jnp.float32),
        ],
        scratch_shapes=[
            pltpu.VMEM((NSEND, S_OUT, CB), jnp.float32),
            pltpu.SemaphoreType.DMA((NSEND,)),
            pltpu.SemaphoreType.DMA((NSEND,)),
            pltpu.SemaphoreType.DMA((NSEND,)),
            pltpu.SemaphoreType.DMA((NSEND,)),
        ],
        compiler_params=pltpu.CompilerParams(
            dimension_semantics=("arbitrary", "arbitrary"),
            vmem_limit_bytes=56 * 1024 * 1024,
            collective_id=0),
    )(a2, Wo)

    out = pl.pallas_call(
        _add_body,
        grid=(NB,),
        in_specs=[
            pl.BlockSpec((S_OUT, CB), lambda j: (0, j)),
            pl.BlockSpec((S_OUT, CB), lambda j: (0, j)),
        ],
        out_specs=pl.BlockSpec((S_OUT, CB), lambda j: (0, j)),
        out_shape=jax.ShapeDtypeStruct((S_OUT, N), jnp.float32),
    )(p, po)
    return out.reshape(1, S_OUT, N)


# device time: 294023 ns/iter; 4.6478x vs baseline; 1.1478x over previous
import jax
import jax.numpy as jnp
from jax import lax
from jax.experimental import pallas as pl
from jax.experimental.pallas import tpu as pltpu

S = 2048
N = 8192
K_LOC = 4096
S_OUT = 1024
CB = 1024
HALF = CB // 2
NB = N // CB
BK = 1024
KB = K_LOC // BK
NSEND = NB // 2

MESH = pl.DeviceIdType.MESH
COMM = True


def _fused_body(a_ref, b_ref, p_ref, po_ref,
                stage, x_send, x_recv, y_send, y_recv):
    n = pl.program_id(0)
    k = pl.program_id(1)
    my_x = lax.axis_index("x")
    my_y = lax.axis_index("y")
    x_partner = (1 - my_x, my_y)
    y_partner = (my_x, 1 - my_y)

    if COMM:
        @pl.when((n == 0) & (k == 0))
        def _():
            bar = pltpu.get_barrier_semaphore()
            pl.semaphore_signal(bar, inc=1, device_id=x_partner,
                                device_id_type=MESH)
            pl.semaphore_signal(bar, inc=1, device_id=y_partner,
                                device_id_type=MESH)
            pl.semaphore_wait(bar, 2)

    @pl.when(k == 0)
    def _():
        p_ref[...] = jnp.zeros_like(p_ref)
        stage[n] = jnp.zeros_like(stage[n])

    a_keep = a_ref[pl.ds(my_x * S_OUT, S_OUT), :]
    p_ref[...] += lax.dot_general(
        a_keep, b_ref[...], (((1,), (0,)), ((), ())),
        preferred_element_type=jnp.float32)

    a_send = a_ref[pl.ds((1 - my_x) * S_OUT, S_OUT), :]
    b_half = b_ref[:, pl.ds(my_y * HALF, HALF)]
    stage[n] += lax.dot_general(
        a_send, b_half, (((1,), (0,)), ((), ())),
        preferred_element_type=jnp.float32)

    def x_desc(m):
        return pltpu.make_async_remote_copy(
            src_ref=stage.at[m],
            dst_ref=po_ref.at[:, pl.ds(m * CB + my_y * HALF, HALF)],
            send_sem=x_send.at[m],
            recv_sem=x_recv.at[m],
            device_id=x_partner,
            device_id_type=MESH)

    def _y_desc(m, col_y):
        return pltpu.make_async_remote_copy(
            src_ref=po_ref.at[:, pl.ds(m * CB + col_y * HALF, HALF)],
            dst_ref=po_ref.at[:, pl.ds(m * CB + col_y * HALF, HALF)],
            send_sem=y_send.at[m],
            recv_sem=y_recv.at[m],
            device_id=y_partner,
            device_id_type=MESH)

    def y_out_desc(m):
        return _y_desc(m, my_y)

    def y_in_desc(m):
        return _y_desc(m, 1 - my_y)

    k_last = KB - 1

    if not COMM:
        return

    @pl.when(k == k_last)
    def _():
        x_desc(n).start()

    def _forward(m):
        x_desc(m).wait_recv()
        y_out_desc(m).start()

    @pl.when((k == 1) & (n >= 2))
    def _():
        _forward(n - 2)

    @pl.when((k == k_last) & (n == NB - 1))
    def _():
        _forward(NB - 2)
        _forward(NB - 1)
        for t in range(NB):
            x_desc(t).wait_send()
        for t in range(NB):
            y_out_desc(t).wait_send()
        for t in range(NB):
            y_in_desc(t).wait_recv()


def _relayout_body(o_ref, a2_ref):
    sb = o_ref.shape[0]
    a2_ref[...] = o_ref[...].reshape(sb, K_LOC)


def _add_body(p_ref, po_ref, out_ref):
    out_ref[...] = p_ref[...] + po_ref[...]


def kernel(O, Wo):
    b, s, h, d = O.shape
    o3 = O.reshape(s, h, d)
    a2 = pl.pallas_call(
        _relayout_body,
        grid=(8,),
        in_specs=[pl.BlockSpec((S // 8, h, d), lambda i: (i, 0, 0))],
        out_specs=pl.BlockSpec((S // 8, K_LOC), lambda i: (i, 0)),
        out_shape=jax.ShapeDtypeStruct((S, K_LOC), jnp.float32),
    )(o3)

    p, po = pl.pallas_call(
        _fused_body,
        grid=(NB, KB),
        in_specs=[
            pl.BlockSpec((S, BK), lambda n, k: (0, k)),
            pl.BlockSpec((BK, CB), lambda n, k: (k, n)),
        ],
        out_specs=[
            pl.BlockSpec((S_OUT, CB), lambda n, k: (0, n)),
            pl.BlockSpec(memory_space=pl.ANY),
        ],
        out_shape=[
            jax.ShapeDtypeStruct((S_OUT, N), jnp.float32),
            jax.ShapeDtypeStruct((S_OUT, N), jnp.float32),
        ],
        scratch_shapes=[
            pltpu.VMEM((NB, S_OUT, HALF), jnp.float32),
            pltpu.SemaphoreType.DMA((NB,)),
            pltpu.SemaphoreType.DMA((NB,)),
            pltpu.SemaphoreType.DMA((NB,)),
            pltpu.SemaphoreType.DMA((NB,)),
        ],
        compiler_params=pltpu.CompilerParams(
            dimension_semantics=("arbitrary", "arbitrary"),
            vmem_limit_bytes=56 * 1024 * 1024,
            collective_id=0 if COMM else None),
    )(a2, Wo)

    out = pl.pallas_call(
        _add_body,
        grid=(NB,),
        in_specs=[
            pl.BlockSpec((S_OUT, CB), lambda j: (0, j)),
            pl.BlockSpec((S_OUT, CB), lambda j: (0, j)),
        ],
        out_specs=pl.BlockSpec((S_OUT, CB), lambda j: (0, j)),
        out_shape=jax.ShapeDtypeStruct((S_OUT, N), jnp.float32),
    )(p, po)
    return out.reshape(1, S_OUT, N)


# device time: 240291 ns/iter; 5.6871x vs baseline; 1.2236x over previous
import jax
import jax.numpy as jnp
from jax import lax
from jax.experimental import pallas as pl
from jax.experimental.pallas import tpu as pltpu

S = 2048
N = 8192
K_LOC = 4096
S_OUT = 1024
CB = 1024
HALF = CB // 2
NB = N // CB
BK = 1024
KB = K_LOC // BK
NSEND = NB // 2

MESH = pl.DeviceIdType.MESH
COMM = True


def _fused_body(a_ref, b_ref, p_ref, po_ref,
                stage, acc, x_send, x_recv, y_send, y_recv):
    n = pl.program_id(0)
    k = pl.program_id(1)
    my_x = lax.axis_index("x")
    my_y = lax.axis_index("y")
    x_partner = (1 - my_x, my_y)
    y_partner = (my_x, 1 - my_y)

    if COMM:
        @pl.when((n == 0) & (k == 0))
        def _():
            bar = pltpu.get_barrier_semaphore()
            pl.semaphore_signal(bar, inc=1, device_id=x_partner,
                                device_id_type=MESH)
            pl.semaphore_signal(bar, inc=1, device_id=y_partner,
                                device_id_type=MESH)
            pl.semaphore_wait(bar, 2)

    @pl.when(k == 0)
    def _():
        p_ref[...] = jnp.zeros_like(p_ref)
        acc[...] = jnp.zeros_like(acc)

    a_keep = a_ref[pl.ds(my_x * S_OUT, S_OUT), :]
    p_ref[...] += lax.dot_general(
        a_keep, b_ref[...], (((1,), (0,)), ((), ())),
        preferred_element_type=jnp.float32)

    a_send = a_ref[pl.ds((1 - my_x) * S_OUT, S_OUT), :]
    b_half = b_ref[:, pl.ds(my_y * HALF, HALF)]
    acc[...] += lax.dot_general(
        a_send, b_half, (((1,), (0,)), ((), ())),
        preferred_element_type=jnp.float32)

    def x_desc(m):
        return pltpu.make_async_remote_copy(
            src_ref=stage.at[m],
            dst_ref=po_ref.at[:, pl.ds(m * CB + my_y * HALF, HALF)],
            send_sem=x_send.at[m],
            recv_sem=x_recv.at[m],
            device_id=x_partner,
            device_id_type=MESH)

    def _y_desc(m, col_y):
        return pltpu.make_async_remote_copy(
            src_ref=po_ref.at[:, pl.ds(m * CB + col_y * HALF, HALF)],
            dst_ref=po_ref.at[:, pl.ds(m * CB + col_y * HALF, HALF)],
            send_sem=y_send.at[m],
            recv_sem=y_recv.at[m],
            device_id=y_partner,
            device_id_type=MESH)

    def y_out_desc(m):
        return _y_desc(m, my_y)

    def y_in_desc(m):
        return _y_desc(m, 1 - my_y)

    k_last = KB - 1

    if not COMM:
        return

    @pl.when(k == k_last)
    def _():
        stage[n] = acc[...].astype(jnp.bfloat16)
        x_desc(n).start()

    def _forward(m):
        x_desc(m).wait_recv()
        y_out_desc(m).start()

    @pl.when((k == 1) & (n >= 2))
    def _():
        _forward(n - 2)

    @pl.when((k == k_last) & (n == NB - 1))
    def _():
        _forward(NB - 2)
        _forward(NB - 1)
        for t in range(NB):
            x_desc(t).wait_send()
        for t in range(NB):
            y_out_desc(t).wait_send()
        for t in range(NB):
            y_in_desc(t).wait_recv()


def _relayout_body(o_ref, a2_ref):
    sb = o_ref.shape[0]
    a2_ref[...] = o_ref[...].reshape(sb, K_LOC)


def _add_body(p_ref, po_ref, out_ref):
    out_ref[...] = p_ref[...] + po_ref[...].astype(jnp.float32)


def kernel(O, Wo):
    b, s, h, d = O.shape
    o3 = O.reshape(s, h, d)
    a2 = pl.pallas_call(
        _relayout_body,
        grid=(8,),
        in_specs=[pl.BlockSpec((S // 8, h, d), lambda i: (i, 0, 0))],
        out_specs=pl.BlockSpec((S // 8, K_LOC), lambda i: (i, 0)),
        out_shape=jax.ShapeDtypeStruct((S, K_LOC), jnp.float32),
    )(o3)

    p, po = pl.pallas_call(
        _fused_body,
        grid=(NB, KB),
        in_specs=[
            pl.BlockSpec((S, BK), lambda n, k: (0, k)),
            pl.BlockSpec((BK, CB), lambda n, k: (k, n)),
        ],
        out_specs=[
            pl.BlockSpec((S_OUT, CB), lambda n, k: (0, n)),
            pl.BlockSpec(memory_space=pl.ANY),
        ],
        out_shape=[
            jax.ShapeDtypeStruct((S_OUT, N), jnp.float32),
            jax.ShapeDtypeStruct((S_OUT, N), jnp.bfloat16),
        ],
        scratch_shapes=[
            pltpu.VMEM((NB, S_OUT, HALF), jnp.bfloat16),
            pltpu.VMEM((S_OUT, HALF), jnp.float32),
            pltpu.SemaphoreType.DMA((NB,)),
            pltpu.SemaphoreType.DMA((NB,)),
            pltpu.SemaphoreType.DMA((NB,)),
            pltpu.SemaphoreType.DMA((NB,)),
        ],
        compiler_params=pltpu.CompilerParams(
            dimension_semantics=("arbitrary", "arbitrary"),
            vmem_limit_bytes=56 * 1024 * 1024,
            collective_id=0 if COMM else None),
    )(a2, Wo)

    out = pl.pallas_call(
        _add_body,
        grid=(NB,),
        in_specs=[
            pl.BlockSpec((S_OUT, CB), lambda j: (0, j)),
            pl.BlockSpec((S_OUT, CB), lambda j: (0, j)),
        ],
        out_specs=pl.BlockSpec((S_OUT, CB), lambda j: (0, j)),
        out_shape=jax.ShapeDtypeStruct((S_OUT, N), jnp.float32),
    )(p, po)
    return out.reshape(1, S_OUT, N)
